# speedup vs baseline: 3.7034x; 3.7034x over previous
import functools

import jax
import jax.numpy as jnp
from jax import lax
from jax.experimental import pallas as pl
from jax.experimental.pallas import tpu as pltpu
from jax.experimental.pallas import tpu_sc as plsc

VOCAB = 100000
EMBED = 128
SEQ = 10
HIDDEN = 1024
BATCH = 16384
NUM_IDX = BATCH * SEQ

NC, NS = 2, 16
NW = NC * NS
B_PER_W = NUM_IDX // NW
GCHUNK = 512

CHUNK_M = 512


def _gather_body(table_hbm, idx_hbm, out_hbm, idx_v, rows_v, sem):
    wid = lax.axis_index("s") * NC + lax.axis_index("c")
    base = wid * B_PER_W

    @pl.loop(0, B_PER_W, step=GCHUNK)
    def _(off):
        pltpu.sync_copy(idx_hbm.at[pl.ds(base + off, GCHUNK)], idx_v)
        pltpu.async_copy(table_hbm.at[idx_v], rows_v, sem).wait()
        pltpu.sync_copy(rows_v, out_hbm.at[pl.ds(base + off, GCHUNK)])


def _sc_gather(table, idx):
    mesh = plsc.VectorSubcoreMesh(core_axis_name="c", subcore_axis_name="s")
    kfn = pl.kernel(
        _gather_body,
        mesh=mesh,
        out_type=jax.ShapeDtypeStruct((NUM_IDX, EMBED), table.dtype),
        scratch_types=[
            pltpu.VMEM((GCHUNK,), jnp.int32),
            pltpu.VMEM((GCHUNK, EMBED), table.dtype),
            pltpu.SemaphoreType.DMA,
        ],
    )
    return kfn(table, idx)


def _mlp_body(flat_ref, w1t_ref, b1_ref, w2_ref, b2_ref, out_ref):
    a = flat_ref[...].astype(jnp.bfloat16)
    h = jnp.dot(a, w1t_ref[...], preferred_element_type=jnp.float32)
    h = jnp.maximum(h + b1_ref[...], 0.0)
    s = jnp.sum(h * w2_ref[...], axis=1) + b2_ref[0, 0]
    out_ref[...] = jax.nn.sigmoid(s)


def _tc_mlp(flat, w1t, b1, w2, b2):
    return pl.pallas_call(
        _mlp_body,
        grid=(BATCH // CHUNK_M,),
        in_specs=[
            pl.BlockSpec((CHUNK_M, SEQ * EMBED), lambda i: (i, 0)),
            pl.BlockSpec((SEQ * EMBED, HIDDEN), lambda i: (0, 0)),
            pl.BlockSpec((1, HIDDEN), lambda i: (0, 0)),
            pl.BlockSpec((1, HIDDEN), lambda i: (0, 0)),
            pl.BlockSpec((1, 1), lambda i: (0, 0)),
        ],
        out_specs=pl.BlockSpec((CHUNK_M,), lambda i: (i,)),
        out_shape=jax.ShapeDtypeStruct((BATCH,), jnp.float32),
        compiler_params=pltpu.CompilerParams(
            dimension_semantics=("arbitrary",),
        ),
    )(flat, w1t, b1, w2, b2)


def kernel(x, table, W1, b1, W2, b2):
    idx = x.reshape(-1)
    rows = _sc_gather(table, idx)
    flat = rows.reshape(BATCH, SEQ * EMBED)
    w1t = W1.T.astype(jnp.bfloat16)
    return _tc_mlp(
        flat,
        w1t,
        b1.reshape(1, HIDDEN),
        W2.reshape(1, HIDDEN),
        b2.reshape(1, 1),
    )


# speedup vs baseline: 4.0334x; 1.0891x over previous
import functools

import jax
import jax.numpy as jnp
from jax import lax
from jax.experimental import pallas as pl
from jax.experimental.pallas import tpu as pltpu
from jax.experimental.pallas import tpu_sc as plsc

VOCAB = 100000
EMBED = 128
SEQ = 10
HIDDEN = 1024
BATCH = 16384
NUM_IDX = BATCH * SEQ

NC, NS = 2, 16
NW = NC * NS

NSLICE = 4
BATCH_S = BATCH // NSLICE
IDX_S = BATCH_S * SEQ
B_PER_W = IDX_S // NW
GCHUNK = 640

CHUNK_M = 512


def _gather_body(table_hbm, idx_hbm, out_hbm, idx_v, rows_v, sem):
    wid = lax.axis_index("s") * NC + lax.axis_index("c")
    base = wid * B_PER_W

    @pl.loop(0, B_PER_W, step=GCHUNK)
    def _(off):
        pltpu.sync_copy(idx_hbm.at[pl.ds(base + off, GCHUNK)], idx_v)
        pltpu.async_copy(table_hbm.at[idx_v], rows_v, sem).wait()
        pltpu.sync_copy(rows_v, out_hbm.at[pl.ds(base + off, GCHUNK)])


def _sc_gather(table, idx):
    mesh = plsc.VectorSubcoreMesh(core_axis_name="c", subcore_axis_name="s")
    kfn = pl.kernel(
        _gather_body,
        mesh=mesh,
        out_type=jax.ShapeDtypeStruct((IDX_S, EMBED), table.dtype),
        scratch_types=[
            pltpu.VMEM((GCHUNK,), jnp.int32),
            pltpu.VMEM((GCHUNK, EMBED), table.dtype),
            pltpu.SemaphoreType.DMA,
        ],
    )
    return kfn(table, idx)


def _mlp_body(flat_ref, w1t_ref, b1_ref, w2_ref, b2_ref, out_ref):
    a = flat_ref[...].astype(jnp.bfloat16)
    h = jnp.dot(a, w1t_ref[...], preferred_element_type=jnp.float32)
    h = jnp.maximum(h + b1_ref[...], 0.0)
    s = jnp.sum(h * w2_ref[...], axis=1) + b2_ref[0, 0]
    out_ref[...] = jax.nn.sigmoid(s)


def _tc_mlp(flat, w1t, b1, w2, b2):
    return pl.pallas_call(
        _mlp_body,
        grid=(BATCH_S // CHUNK_M,),
        in_specs=[
            pl.BlockSpec((CHUNK_M, SEQ * EMBED), lambda i: (i, 0)),
            pl.BlockSpec((SEQ * EMBED, HIDDEN), lambda i: (0, 0)),
            pl.BlockSpec((1, HIDDEN), lambda i: (0, 0)),
            pl.BlockSpec((1, HIDDEN), lambda i: (0, 0)),
            pl.BlockSpec((1, 1), lambda i: (0, 0)),
        ],
        out_specs=pl.BlockSpec((CHUNK_M,), lambda i: (i,)),
        out_shape=jax.ShapeDtypeStruct((BATCH_S,), jnp.float32),
        compiler_params=pltpu.CompilerParams(
            dimension_semantics=("arbitrary",),
        ),
    )(flat, w1t, b1, w2, b2)


def kernel(x, table, W1, b1, W2, b2):
    w1t = W1.T.astype(jnp.bfloat16)
    b1r = b1.reshape(1, HIDDEN)
    w2r = W2.reshape(1, HIDDEN)
    b2r = b2.reshape(1, 1)
    outs = []
    for s in range(NSLICE):
        idx = x[s * BATCH_S:(s + 1) * BATCH_S].reshape(-1)
        rows = _sc_gather(table, idx)
        flat = rows.reshape(BATCH_S, SEQ * EMBED)
        outs.append(_tc_mlp(flat, w1t, b1r, w2r, b2r))
    return jnp.concatenate(outs)


# speedup vs baseline: 6.9507x; 1.7233x over previous
import functools

import jax
import jax.numpy as jnp
from jax import lax
from jax.experimental import pallas as pl
from jax.experimental.pallas import tpu as pltpu
from jax.experimental.pallas import tpu_sc as plsc

VOCAB = 100000
EMBED = 128
SEQ = 10
HIDDEN = 1024
BATCH = 16384
NUM_IDX = BATCH * SEQ

NC, NS = 2, 16
NW = NC * NS

NSLICE = 4
BATCH_S = BATCH // NSLICE
IDX_S = BATCH_S * SEQ
B_PER_W = IDX_S // NW
GCHUNK = 640

CHUNK_M = 512


def _gather_body(table_hbm, idx_hbm, out_hbm, idx_v, rows_v, sem):
    wid = lax.axis_index("s") * NC + lax.axis_index("c")
    base = wid * B_PER_W

    @pl.loop(0, B_PER_W, step=GCHUNK)
    def _(off):
        pltpu.sync_copy(idx_hbm.at[pl.ds(base + off, GCHUNK)], idx_v)
        pltpu.async_copy(table_hbm.at[idx_v], rows_v, sem).wait()
        pltpu.sync_copy(rows_v, out_hbm.at[pl.ds(base + off, GCHUNK)])


def _sc_gather(table, idx):
    mesh = plsc.VectorSubcoreMesh(core_axis_name="c", subcore_axis_name="s")
    kfn = pl.kernel(
        _gather_body,
        mesh=mesh,
        out_type=jax.ShapeDtypeStruct((IDX_S, EMBED), table.dtype),
        scratch_types=[
            pltpu.VMEM((GCHUNK,), jnp.int32),
            pltpu.VMEM((GCHUNK, EMBED), table.dtype),
            pltpu.SemaphoreType.DMA,
        ],
    )
    return kfn(table, idx)


def _mlp_body(g_ref, w1t_ref, b1_ref, w2_ref, b2_ref, out_ref):
    a = jnp.concatenate([g_ref[s] for s in range(SEQ)], axis=-1)
    a = a.astype(jnp.bfloat16)
    h = jnp.dot(a, w1t_ref[...], preferred_element_type=jnp.float32)
    h = jnp.maximum(h + b1_ref[...], 0.0)
    s = jnp.sum(h * w2_ref[...], axis=1) + b2_ref[0, 0]
    out_ref[...] = jax.nn.sigmoid(s)


def _tc_mlp(g3, w1t, b1, w2, b2):
    return pl.pallas_call(
        _mlp_body,
        grid=(BATCH_S // CHUNK_M,),
        in_specs=[
            pl.BlockSpec((SEQ, CHUNK_M, EMBED), lambda i: (0, i, 0)),
            pl.BlockSpec((SEQ * EMBED, HIDDEN), lambda i: (0, 0)),
            pl.BlockSpec((1, HIDDEN), lambda i: (0, 0)),
            pl.BlockSpec((1, HIDDEN), lambda i: (0, 0)),
            pl.BlockSpec((1, 1), lambda i: (0, 0)),
        ],
        out_specs=pl.BlockSpec((CHUNK_M,), lambda i: (i,)),
        out_shape=jax.ShapeDtypeStruct((BATCH_S,), jnp.float32),
        compiler_params=pltpu.CompilerParams(
            dimension_semantics=("arbitrary",),
        ),
    )(g3, w1t, b1, w2, b2)


def kernel(x, table, W1, b1, W2, b2):
    w1t = W1.T.astype(jnp.bfloat16)
    b1r = b1.reshape(1, HIDDEN)
    w2r = W2.reshape(1, HIDDEN)
    b2r = b2.reshape(1, 1)
    outs = []
    for s in range(NSLICE):
        xs = x[s * BATCH_S:(s + 1) * BATCH_S]
        idx = xs.T.reshape(-1)
        rows = _sc_gather(table, idx)
        g3 = rows.reshape(SEQ, BATCH_S, EMBED)
        outs.append(_tc_mlp(g3, w1t, b1r, w2r, b2r))
    return jnp.concatenate(outs)


# speedup vs baseline: 7.0945x; 1.0207x over previous
import functools

import jax
import jax.numpy as jnp
from jax import lax
from jax.experimental import pallas as pl
from jax.experimental.pallas import tpu as pltpu
from jax.experimental.pallas import tpu_sc as plsc

VOCAB = 100000
EMBED = 128
SEQ = 10
HIDDEN = 1024
BATCH = 16384
NUM_IDX = BATCH * SEQ

NC, NS = 2, 16
NW = NC * NS

NSLICE = 4
BATCH_S = BATCH // NSLICE
IDX_S = BATCH_S * SEQ
B_PER_W = IDX_S // NW
GCHUNK = 640

CHUNK_M = 1024


def _gather_body(table_hbm, idx_hbm, out_hbm, idx_v, rows_v, sem):
    wid = lax.axis_index("s") * NC + lax.axis_index("c")
    base = wid * B_PER_W

    @pl.loop(0, B_PER_W, step=GCHUNK)
    def _(off):
        pltpu.sync_copy(idx_hbm.at[pl.ds(base + off, GCHUNK)], idx_v)
        pltpu.async_copy(table_hbm.at[idx_v], rows_v, sem).wait()
        pltpu.sync_copy(rows_v, out_hbm.at[pl.ds(base + off, GCHUNK)])


def _sc_gather(table, idx):
    mesh = plsc.VectorSubcoreMesh(core_axis_name="c", subcore_axis_name="s")
    kfn = pl.kernel(
        _gather_body,
        mesh=mesh,
        out_type=jax.ShapeDtypeStruct((IDX_S, EMBED), table.dtype),
        scratch_types=[
            pltpu.VMEM((GCHUNK,), jnp.int32),
            pltpu.VMEM((GCHUNK, EMBED), table.dtype),
            pltpu.SemaphoreType.DMA,
        ],
    )
    return kfn(table, idx)


def _mlp_body(g_ref, w1t_ref, b1_ref, w2c_ref, b2_ref, out_ref):
    a = jnp.concatenate([g_ref[s] for s in range(SEQ)], axis=-1)
    a = a.astype(jnp.bfloat16)
    h = jnp.dot(a, w1t_ref[...], preferred_element_type=jnp.float32)
    h = jnp.maximum(h + b1_ref[...], 0.0)
    s128 = jnp.dot(h.astype(jnp.bfloat16), w2c_ref[...],
                   preferred_element_type=jnp.float32)
    s = s128[:, 0] + b2_ref[0, 0]
    out_ref[...] = jax.nn.sigmoid(s)


def _tc_mlp(g3, w1t, b1, w2c, b2):
    return pl.pallas_call(
        _mlp_body,
        grid=(BATCH_S // CHUNK_M,),
        in_specs=[
            pl.BlockSpec((SEQ, CHUNK_M, EMBED), lambda i: (0, i, 0)),
            pl.BlockSpec((SEQ * EMBED, HIDDEN), lambda i: (0, 0)),
            pl.BlockSpec((1, HIDDEN), lambda i: (0, 0)),
            pl.BlockSpec((HIDDEN, 128), lambda i: (0, 0)),
            pl.BlockSpec((1, 1), lambda i: (0, 0)),
        ],
        out_specs=pl.BlockSpec((CHUNK_M,), lambda i: (i,)),
        out_shape=jax.ShapeDtypeStruct((BATCH_S,), jnp.float32),
        compiler_params=pltpu.CompilerParams(
            dimension_semantics=("arbitrary",),
        ),
    )(g3, w1t, b1, w2c, b2)


def kernel(x, table, W1, b1, W2, b2):
    w1t = W1.T.astype(jnp.bfloat16)
    b1r = b1.reshape(1, HIDDEN)
    w2c = jnp.zeros((HIDDEN, 128), jnp.float32).at[:, 0].set(W2[0])
    w2c = w2c.astype(jnp.bfloat16)
    b2r = b2.reshape(1, 1)
    outs = []
    for s in range(NSLICE):
        xs = x[s * BATCH_S:(s + 1) * BATCH_S]
        idx = xs.T.reshape(-1)
        rows = _sc_gather(table, idx)
        g3 = rows.reshape(SEQ, BATCH_S, EMBED)
        outs.append(_tc_mlp(g3, w1t, b1r, w2c, b2r))
    return jnp.concatenate(outs)
